# baseline (speedup 1.0000x reference)
import functools

import numpy as np
import jax
import jax.numpy as jnp
from jax import lax
from jax.experimental import pallas as pl
from jax.experimental.pallas import tpu as pltpu
from jax.experimental.pallas import tpu_sc as plsc

_TARGET = 0

N_ROWS = 1_000_000
ROW = 8
L = 16
NC, NS = 2, 16
NW = NC * NS
LANE = 128

_CLS, _SCORE = 0, ROW - 1
_ROWS_CONST = np.array([_CLS, _SCORE], dtype=np.int32)

TILES_PER_W = 48
SC_LANES = TILES_PER_W * NW * LANE
CHUNK_LANES = TILES_PER_W * LANE
GROUPS = CHUNK_LANES // L

TC_BLOCK = 32768
TC_START_BLK = SC_LANES // TC_BLOCK
TC_LANES = N_ROWS - SC_LANES
TC_STEPS = -(-TC_LANES // TC_BLOCK)
TC_LAST_VALID = TC_LANES - (TC_STEPS - 1) * TC_BLOCK
ACC_W = 2048


def _sc_partials_body(xt_hbm, rows_hbm, out_hbm, idx_ref, buf, acc_ref, sem):
    cid = lax.axis_index("c")
    sid = lax.axis_index("s")
    wid = sid * NC + cid

    base = pl.multiple_of(wid * CHUNK_LANES, LANE)
    pltpu.sync_copy(rows_hbm, idx_ref)
    cp = pltpu.async_copy(
        xt_hbm.at[idx_ref, pl.ds(base, CHUNK_LANES)], buf, sem
    )
    cp.wait()

    def group_body(g, acc):
        cls = buf[0, pl.ds(g * L, L)]
        sc = buf[1, pl.ds(g * L, L)]
        keep = cls.astype(jnp.int32) == _TARGET
        return acc + jnp.where(keep, sc, 0.0)

    acc = plsc.parallel_loop(
        0, GROUPS, unroll=4, carry=jnp.zeros((L,), jnp.float32)
    )(group_body)
    acc_ref[...] = acc
    pltpu.sync_copy(acc_ref, out_hbm.at[wid])


_sc_partials = pl.kernel(
    _sc_partials_body,
    out_type=jax.ShapeDtypeStruct((NW, L), jnp.float32),
    mesh=plsc.VectorSubcoreMesh(
        core_axis_name="c", subcore_axis_name="s", num_cores=NC, num_subcores=NS
    ),
    compiler_params=pltpu.CompilerParams(
        needs_layout_passes=False, use_tc_tiling_on_sc=True
    ),
    scratch_types=[
        pltpu.VMEM((2,), jnp.int32),
        pltpu.VMEM((2, CHUNK_LANES), jnp.float32),
        pltpu.VMEM((L,), jnp.float32),
        pltpu.SemaphoreType.DMA,
    ],
)


def _fold(p):
    parts = [p[:, j * ACC_W : (j + 1) * ACC_W] for j in range(TC_BLOCK // ACC_W)]
    while len(parts) > 1:
        parts = [
            parts[k] + parts[k + 1] for k in range(0, len(parts) - 1, 2)
        ] + ([parts[-1]] if len(parts) % 2 else [])
    return parts[0]


def _tc_reduce_body(x_ref, o_ref, acc_ref):
    i = pl.program_id(0)

    @pl.when(i == 0)
    def _():
        acc_ref[...] = jnp.zeros_like(acc_ref)

    cls = x_ref[_CLS : _CLS + 1, :]
    sc = x_ref[_SCORE : _SCORE + 1, :]
    p = jnp.where(cls.astype(jnp.int32) == _TARGET, sc, 0.0)

    @pl.when(i < TC_STEPS - 1)
    def _():
        acc_ref[...] += _fold(p)

    @pl.when(i == TC_STEPS - 1)
    def _():
        lanes = lax.broadcasted_iota(jnp.int32, (1, TC_BLOCK), 1)
        acc_ref[...] += _fold(jnp.where(lanes < TC_LAST_VALID, p, 0.0))
        o_ref[0, 0] = jnp.sum(acc_ref[...])


_tc_reduce = pl.pallas_call(
    _tc_reduce_body,
    out_shape=jax.ShapeDtypeStruct((1, 1), jnp.float32),
    grid=(TC_STEPS,),
    in_specs=[
        pl.BlockSpec((ROW, TC_BLOCK), lambda i: (0, TC_START_BLK + i)),
    ],
    out_specs=pl.BlockSpec((1, 1), lambda i: (0, 0), memory_space=pltpu.SMEM),
    scratch_shapes=[pltpu.VMEM((1, ACC_W), jnp.float32)],
)


def _finish_body(p_ref, t_ref, o_ref):
    o_ref[0, 0] = -(jnp.sum(p_ref[...]) + t_ref[0, 0])


_finish = pl.pallas_call(
    _finish_body,
    out_shape=jax.ShapeDtypeStruct((1, 1), jnp.float32),
    in_specs=[
        pl.BlockSpec(memory_space=pltpu.VMEM),
        pl.BlockSpec(memory_space=pltpu.SMEM),
    ],
    out_specs=pl.BlockSpec(memory_space=pltpu.SMEM),
)


@jax.jit
def kernel(box3d_branch):
    xt = box3d_branch.T
    partials = _sc_partials(xt, _ROWS_CONST)
    tc_part = _tc_reduce(xt)
    return _finish(partials, tc_part)[0, 0]


# speedup vs baseline: 1.2825x; 1.2825x over previous
import functools

import numpy as np
import jax
import jax.numpy as jnp
from jax import lax
from jax.experimental import pallas as pl
from jax.experimental.pallas import tpu as pltpu
from jax.experimental.pallas import tpu_sc as plsc

_TARGET = 0

N_ROWS = 1_000_000
ROW = 8
L = 16
NC, NS = 2, 16
NW = NC * NS
LANE = 128

_CLS, _SCORE = 0, ROW - 1
_ROWS_CONST = np.array([_CLS, _SCORE], dtype=np.int32)

TILES_PER_W = 176
SC_LANES = TILES_PER_W * NW * LANE
CHUNK_LANES = TILES_PER_W * LANE
GROUPS = CHUNK_LANES // L

TC_BLOCK = 32768
TC_START_BLK = SC_LANES // TC_BLOCK
TC_LANES = N_ROWS - SC_LANES
TC_STEPS = -(-TC_LANES // TC_BLOCK)
TC_LAST_VALID = TC_LANES - (TC_STEPS - 1) * TC_BLOCK
ACC_W = 2048


def _sc_partials_body(xt_hbm, rows_hbm, out_hbm, idx_ref, buf, acc_ref, sem):
    cid = lax.axis_index("c")
    sid = lax.axis_index("s")
    wid = sid * NC + cid

    base = pl.multiple_of(wid * CHUNK_LANES, LANE)
    pltpu.sync_copy(rows_hbm, idx_ref)
    cp = pltpu.async_copy(
        xt_hbm.at[idx_ref, pl.ds(base, CHUNK_LANES)], buf, sem
    )
    cp.wait()

    def group_body(g, acc):
        cls = buf[0, pl.ds(g * L, L)]
        sc = buf[1, pl.ds(g * L, L)]
        keep = cls.astype(jnp.int32) == _TARGET
        return acc + jnp.where(keep, sc, 0.0)

    acc = plsc.parallel_loop(
        0, GROUPS, unroll=4, carry=jnp.zeros((L,), jnp.float32)
    )(group_body)
    acc_ref[...] = acc
    pltpu.sync_copy(acc_ref, out_hbm.at[wid])


_sc_partials = pl.kernel(
    _sc_partials_body,
    out_type=jax.ShapeDtypeStruct((NW, L), jnp.float32),
    mesh=plsc.VectorSubcoreMesh(
        core_axis_name="c", subcore_axis_name="s", num_cores=NC, num_subcores=NS
    ),
    compiler_params=pltpu.CompilerParams(
        needs_layout_passes=False, use_tc_tiling_on_sc=True
    ),
    scratch_types=[
        pltpu.VMEM((2,), jnp.int32),
        pltpu.VMEM((2, CHUNK_LANES), jnp.float32),
        pltpu.VMEM((L,), jnp.float32),
        pltpu.SemaphoreType.DMA,
    ],
)


def _fold(p):
    parts = [p[:, j * ACC_W : (j + 1) * ACC_W] for j in range(TC_BLOCK // ACC_W)]
    while len(parts) > 1:
        parts = [
            parts[k] + parts[k + 1] for k in range(0, len(parts) - 1, 2)
        ] + ([parts[-1]] if len(parts) % 2 else [])
    return parts[0]


def _tc_reduce_body(x_ref, o_ref, acc_ref):
    i = pl.program_id(0)

    @pl.when(i == 0)
    def _():
        acc_ref[...] = jnp.zeros_like(acc_ref)

    cls = x_ref[_CLS : _CLS + 1, :]
    sc = x_ref[_SCORE : _SCORE + 1, :]
    p = jnp.where(cls.astype(jnp.int32) == _TARGET, sc, 0.0)

    @pl.when(i < TC_STEPS - 1)
    def _():
        acc_ref[...] += _fold(p)

    @pl.when(i == TC_STEPS - 1)
    def _():
        lanes = lax.broadcasted_iota(jnp.int32, (1, TC_BLOCK), 1)
        acc_ref[...] += _fold(jnp.where(lanes < TC_LAST_VALID, p, 0.0))
        o_ref[0, 0] = jnp.sum(acc_ref[...])


_tc_reduce = pl.pallas_call(
    _tc_reduce_body,
    out_shape=jax.ShapeDtypeStruct((1, 1), jnp.float32),
    grid=(TC_STEPS,),
    in_specs=[
        pl.BlockSpec((ROW, TC_BLOCK), lambda i: (0, TC_START_BLK + i)),
    ],
    out_specs=pl.BlockSpec((1, 1), lambda i: (0, 0), memory_space=pltpu.SMEM),
    scratch_shapes=[pltpu.VMEM((1, ACC_W), jnp.float32)],
)


def _finish_body(p_ref, t_ref, o_ref):
    o_ref[0, 0] = -(jnp.sum(p_ref[...]) + t_ref[0, 0])


_finish = pl.pallas_call(
    _finish_body,
    out_shape=jax.ShapeDtypeStruct((1, 1), jnp.float32),
    in_specs=[
        pl.BlockSpec(memory_space=pltpu.VMEM),
        pl.BlockSpec(memory_space=pltpu.SMEM),
    ],
    out_specs=pl.BlockSpec(memory_space=pltpu.SMEM),
)


@jax.jit
def kernel(box3d_branch):
    xt = box3d_branch.T
    partials = _sc_partials(xt, _ROWS_CONST)
    tc_part = _tc_reduce(xt)
    return _finish(partials, tc_part)[0, 0]
